# Initial kernel scaffold; baseline (speedup 1.0000x reference)
import functools

import jax
import jax.numpy as jnp
from jax import lax
from jax.experimental import pallas as pl
from jax.experimental.pallas import tpu as pltpu
from jax.experimental.pallas import tpu_sc as plsc

N = 10000
E = 320000
D = 128
H = 128
O = 128
G = 128
L = 3

NC = 2
NS = 16
NW = NC * NS
CHUNK = 128
NCHUNKS = -(-E // (NW * CHUNK))
EPAD = NW * NCHUNKS * CHUNK
ACC_ROWS = 16 * 640
ROWS_PER_TILE_OUT = N // NS


def _sc_agg_body(x_hbm, src_hbm, dst_hbm, out_hbm, src_v, dst_v, rows_v,
                 acc_sh, sem):
    c = lax.axis_index("c")
    s = lax.axis_index("s")
    wid = s * NC + c

    pltpu.sync_copy(src_hbm.at[wid], src_v)
    pltpu.sync_copy(dst_hbm.at[wid], dst_v)

    zero16 = jnp.zeros((16,), jnp.float32)

    def _zrow(i, _):
        def _zlane(k, _):
            rows_v[i, pl.ds(k * 16, 16)] = zero16
            return 0
        return lax.fori_loop(0, D // 16, _zlane, 0)

    lax.fori_loop(0, CHUNK, _zrow, 0)

    zbase = s * 640
    for t in range(5):
        pltpu.sync_copy(rows_v, acc_sh.at[pl.ds(zbase + t * CHUNK, CHUNK)])

    plsc.subcore_barrier()

    def _chunk(j, _):
        pltpu.async_copy(x_hbm.at[src_v.at[j]], rows_v, sem).wait()
        pltpu.sync_copy(rows_v, acc_sh.at[dst_v.at[j]], add=True)
        return 0

    lax.fori_loop(0, NCHUNKS, _chunk, 0)

    plsc.subcore_barrier()

    r0 = s * ROWS_PER_TILE_OUT
    pltpu.sync_copy(acc_sh.at[pl.ds(r0, ROWS_PER_TILE_OUT)],
                    out_hbm.at[c, pl.ds(r0, ROWS_PER_TILE_OUT)])


_sc_agg = functools.partial(
    pl.kernel,
    out_type=jax.ShapeDtypeStruct((NC, N, D), jnp.float32),
    mesh=plsc.VectorSubcoreMesh(core_axis_name="c", subcore_axis_name="s"),
    scratch_types=[
        pltpu.VMEM((NCHUNKS, CHUNK), jnp.int32),
        pltpu.VMEM((NCHUNKS, CHUNK), jnp.int32),
        pltpu.VMEM((CHUNK, D), jnp.float32),
        pltpu.VMEM_SHARED((ACC_ROWS, D), jnp.float32),
        pltpu.SemaphoreType.DMA,
    ],
)(_sc_agg_body)


def _mlp_body(agg_ref, x_ref, w1_ref, b1_ref, g_ref, be_ref, w2_ref, b2_ref,
              o_ref):
    h = agg_ref[0] + agg_ref[1] + x_ref[...]
    h = lax.dot_general(h, w1_ref[...], (((1,), (0,)), ((), ())),
                        preferred_element_type=jnp.float32,
                        precision=lax.Precision.HIGHEST) + b1_ref[...]
    mu = jnp.mean(h, axis=0, keepdims=True)
    var = jnp.mean(jnp.square(h - mu), axis=0, keepdims=True)
    h = (h - mu) / jnp.sqrt(var + 1e-5) * g_ref[...] + be_ref[...]
    h = jnp.maximum(h, 0.0)
    h = lax.dot_general(h, w2_ref[...], (((1,), (0,)), ((), ())),
                        preferred_element_type=jnp.float32,
                        precision=lax.Precision.HIGHEST) + b2_ref[...]
    o_ref[...] = jnp.maximum(h, 0.0)


def _mlp_call(agg, x, w1, b1, g, be, w2, b2):
    return pl.pallas_call(
        _mlp_body,
        out_shape=jax.ShapeDtypeStruct((N, H), jnp.float32),
    )(agg, x, w1, b1, g, be, w2, b2)


def _pool_body(x_ref, batch_ref, w1_ref, b1_ref, w2_ref, b2_ref, o_ref):
    gi = lax.broadcasted_iota(jnp.int32, (G, N), 0)
    oh = jnp.where(batch_ref[...] == gi, 1.0, 0.0).astype(jnp.float32)
    pooled = lax.dot_general(oh, x_ref[...], (((1,), (0,)), ((), ())),
                             preferred_element_type=jnp.float32,
                             precision=lax.Precision.HIGHEST)
    h = lax.dot_general(pooled, w1_ref[...], (((1,), (0,)), ((), ())),
                        preferred_element_type=jnp.float32,
                        precision=lax.Precision.HIGHEST) + b1_ref[...]
    h = jnp.maximum(h, 0.0)
    o_ref[...] = lax.dot_general(h, w2_ref[...], (((1,), (0,)), ((), ())),
                                 preferred_element_type=jnp.float32,
                                 precision=lax.Precision.HIGHEST) + b2_ref[...]


def _pool_call(x, batch_row, w1, b1, w2, b2):
    return pl.pallas_call(
        _pool_body,
        out_shape=jax.ShapeDtypeStruct((G, O), jnp.float32),
    )(x, batch_row, w1, b1, w2, b2)


def kernel(x, edge_index, batch, conv_W1, conv_b1, conv_gamma, conv_beta,
           conv_W2, conv_b2, mlp_W1, mlp_b1, mlp_W2, mlp_b2):
    src = edge_index[0]
    dst = edge_index[1]
    pad = EPAD - E
    src_p = jnp.concatenate([src, jnp.zeros((pad,), jnp.int32)])
    dst_p = jnp.concatenate([dst, jnp.full((pad,), N, jnp.int32)])
    src3 = src_p.reshape(NW, NCHUNKS, CHUNK)
    dst3 = dst_p.reshape(NW, NCHUNKS, CHUNK)
    batch_row = batch.reshape(1, N)

    for l in range(L):
        parts = _sc_agg(x, src3, dst3)
        x = _mlp_call(parts, x, conv_W1[l], conv_b1[l].reshape(1, H),
                      conv_gamma[l].reshape(1, H), conv_beta[l].reshape(1, H),
                      conv_W2[l], conv_b2[l].reshape(1, H))

    return _pool_call(x, batch_row, mlp_W1, mlp_b1.reshape(1, H),
                      mlp_W2, mlp_b2.reshape(1, O))


# speedup vs baseline: 4.1113x; 4.1113x over previous
import functools

import jax
import jax.numpy as jnp
from jax import lax
from jax.experimental import pallas as pl
from jax.experimental.pallas import tpu as pltpu
from jax.experimental.pallas import tpu_sc as plsc

N = 10000
E = 320000
D = 128
H = 128
O = 128
G = 128
L = 3

NC = 2
NS = 16
NW = NC * NS
CHUNK = 128
NCHUNKS = -(-E // (NW * CHUNK))
EPAD = NW * NCHUNKS * CHUNK
ACC_ROWS = 16 * 640
ROWS_PER_TILE_OUT = 624


def _sc_agg_body(x_hbm, src_hbm, dst_hbm, out_hbm, src_v, dst_v, rows_v,
                 acc_sh, sem):
    c = lax.axis_index("c")
    s = lax.axis_index("s")
    wid = s * NC + c

    pltpu.sync_copy(src_hbm.at[wid], src_v)
    pltpu.sync_copy(dst_hbm.at[wid], dst_v)

    zero16 = jnp.zeros((16,), jnp.float32)

    def _zrow(i, _):
        def _zlane(k, _):
            rows_v[i, pl.ds(k * 16, 16)] = zero16
            return 0
        return lax.fori_loop(0, D // 16, _zlane, 0)

    lax.fori_loop(0, CHUNK, _zrow, 0)

    zbase = s * 640
    for t in range(5):
        pltpu.sync_copy(rows_v, acc_sh.at[pl.ds(zbase + t * CHUNK, CHUNK)])

    plsc.subcore_barrier()

    def _chunk(j, _):
        pltpu.async_copy(x_hbm.at[src_v.at[j]], rows_v, sem).wait()
        pltpu.sync_copy(rows_v, acc_sh.at[dst_v.at[j]], add=True)
        return 0

    lax.fori_loop(0, NCHUNKS, _chunk, 0)

    plsc.subcore_barrier()

    r0 = s * ROWS_PER_TILE_OUT
    pltpu.sync_copy(acc_sh.at[pl.ds(r0, ROWS_PER_TILE_OUT)],
                    out_hbm.at[c, pl.ds(r0, ROWS_PER_TILE_OUT)])

    tail = NS * ROWS_PER_TILE_OUT

    @pl.when(s == 0)
    def _emit_tail():
        pltpu.sync_copy(acc_sh.at[pl.ds(tail, N - tail)],
                        out_hbm.at[c, pl.ds(tail, N - tail)])


@functools.lru_cache(maxsize=1)
def _sc_agg_kernel():
    return functools.partial(
        pl.kernel,
        out_type=jax.ShapeDtypeStruct((NC, N, D), jnp.float32),
        mesh=plsc.VectorSubcoreMesh(core_axis_name="c", subcore_axis_name="s"),
        scratch_types=[
            pltpu.VMEM((NCHUNKS, CHUNK), jnp.int32),
            pltpu.VMEM((NCHUNKS, CHUNK), jnp.int32),
            pltpu.VMEM((CHUNK, D), jnp.float32),
            pltpu.VMEM_SHARED((ACC_ROWS, D), jnp.float32),
            pltpu.SemaphoreType.DMA,
        ],
    )(_sc_agg_body)


def _mlp_body(agg_ref, x_ref, w1_ref, b1_ref, g_ref, be_ref, w2_ref, b2_ref,
              o_ref):
    h = agg_ref[0] + agg_ref[1] + x_ref[...]
    h = lax.dot_general(h, w1_ref[...], (((1,), (0,)), ((), ())),
                        preferred_element_type=jnp.float32,
                        precision=lax.Precision.HIGHEST) + b1_ref[...]
    mu = jnp.mean(h, axis=0, keepdims=True)
    var = jnp.mean(jnp.square(h - mu), axis=0, keepdims=True)
    h = (h - mu) / jnp.sqrt(var + 1e-5) * g_ref[...] + be_ref[...]
    h = jnp.maximum(h, 0.0)
    h = lax.dot_general(h, w2_ref[...], (((1,), (0,)), ((), ())),
                        preferred_element_type=jnp.float32,
                        precision=lax.Precision.HIGHEST) + b2_ref[...]
    o_ref[...] = jnp.maximum(h, 0.0)


def _mlp_call(agg, x, w1, b1, g, be, w2, b2):
    return pl.pallas_call(
        _mlp_body,
        out_shape=jax.ShapeDtypeStruct((N, H), jnp.float32),
    )(agg, x, w1, b1, g, be, w2, b2)


def _pool_body(x_ref, batch_ref, w1_ref, b1_ref, w2_ref, b2_ref, o_ref):
    gi = lax.broadcasted_iota(jnp.int32, (G, N), 0)
    oh = jnp.where(batch_ref[...] == gi, 1.0, 0.0).astype(jnp.float32)
    pooled = lax.dot_general(oh, x_ref[...], (((1,), (0,)), ((), ())),
                             preferred_element_type=jnp.float32,
                             precision=lax.Precision.HIGHEST)
    h = lax.dot_general(pooled, w1_ref[...], (((1,), (0,)), ((), ())),
                        preferred_element_type=jnp.float32,
                        precision=lax.Precision.HIGHEST) + b1_ref[...]
    h = jnp.maximum(h, 0.0)
    o_ref[...] = lax.dot_general(h, w2_ref[...], (((1,), (0,)), ((), ())),
                                 preferred_element_type=jnp.float32,
                                 precision=lax.Precision.HIGHEST) + b2_ref[...]


def _pool_call(x, batch_row, w1, b1, w2, b2):
    return pl.pallas_call(
        _pool_body,
        out_shape=jax.ShapeDtypeStruct((G, O), jnp.float32),
    )(x, batch_row, w1, b1, w2, b2)


def kernel(x, edge_index, batch, conv_W1, conv_b1, conv_gamma, conv_beta,
           conv_W2, conv_b2, mlp_W1, mlp_b1, mlp_W2, mlp_b2):
    src = edge_index[0]
    dst = edge_index[1]
    pad = EPAD - E
    src_p = jnp.concatenate([src, jnp.zeros((pad,), jnp.int32)])
    dst_p = jnp.concatenate([dst, jnp.full((pad,), N, jnp.int32)])
    src3 = src_p.reshape(NW, NCHUNKS, CHUNK)
    dst3 = dst_p.reshape(NW, NCHUNKS, CHUNK)
    batch_row = batch.reshape(1, N)

    for l in range(L):
        parts = _sc_agg_kernel()(x, src3, dst3)
        x = _mlp_call(parts, x, conv_W1[l], conv_b1[l].reshape(1, H),
                      conv_gamma[l].reshape(1, H), conv_beta[l].reshape(1, H),
                      conv_W2[l], conv_b2[l].reshape(1, H))

    return _pool_call(x, batch_row, mlp_W1, mlp_b1.reshape(1, H),
                      mlp_W2, mlp_b2.reshape(1, O))
